# baseline (speedup 1.0000x reference)
import functools

import jax
import jax.numpy as jnp
from jax import lax
from jax.experimental import pallas as pl
from jax.experimental.pallas import tpu as pltpu
from jax.experimental.pallas import tpu_sc as plsc

N = 10000
E = 160000
F_IN = 4
H = 512

NT = 16
NCORES = 2

NPAD4 = 10240
EPAD = 160256
EN = EPAD // NT
ECH = EN // 2
GCH = ECH // 16
MCH = ECH * 4
EB = EPAD // 32
GB = EB // 16

RR = 157
NR = 64
NPADW = RR * NR
ACC1 = RR * H
SLOTR = 320

_mesh = plsc.VectorSubcoreMesh(core_axis_name="c", subcore_axis_name="s")
_CP = pltpu.CompilerParams(needs_layout_passes=False)


_GDN = lax.GatherDimensionNumbers(
    offset_dims=(), collapsed_slice_dims=(0,), start_index_map=(0,))


def _blane(v, e):
    idx = jnp.full((16, 1), e, jnp.int32)
    return lax.gather(v, idx, _GDN, (1,),
                      mode=lax.GatherScatterMode.PROMISE_IN_BOUNDS)


def _rsqrt16(x):
    i = lax.bitcast_convert_type(x, jnp.int32)
    i = jnp.int32(0x5F3759DF) - lax.shift_right_logical(i, 1)
    y = lax.bitcast_convert_type(i, jnp.float32)
    for _ in range(3):
        y = y * (1.5 - 0.5 * x * y * y)
    return y


@functools.partial(
    pl.kernel,
    out_type=jax.ShapeDtypeStruct((NPAD4,), jnp.float32),
    mesh=_mesh,
    compiler_params=_CP,
    scratch_types=[
        pltpu.VMEM((EN,), jnp.int32),
        pltpu.VMEM((EN,), jnp.float32),
        pltpu.VMEM((640,), jnp.float32),
        pltpu.VMEM_SHARED((NPAD4,), jnp.float32),
    ],
)
def _deg_dis_kernel(col_hbm, w_hbm, zeros_hbm, dis_hbm, colv, wv, dvm, deg):
    c = lax.axis_index("c")
    s = lax.axis_index("s")

    @pl.when(c == 0)
    def _():
        pltpu.sync_copy(col_hbm.at[s], colv)
        pltpu.sync_copy(w_hbm.at[s], wv)
        pltpu.sync_copy(zeros_hbm.at[pl.ds(s * 640, 640)],
                        deg.at[pl.ds(s * 640, 640)])
        plsc.subcore_barrier()
        pltpu.sync_copy(wv, deg.at[colv], add=True)
        plsc.subcore_barrier()
        pltpu.sync_copy(deg.at[pl.ds(s * 640, 640)], dvm)

        def body(g, carry):
            d16 = dvm[pl.ds(g * 16, 16)]
            y = _rsqrt16(d16)
            dvm[pl.ds(g * 16, 16)] = jnp.where(d16 > 0.0, y, 0.0)
            return carry

        lax.fori_loop(0, 40, body, 0)
        pltpu.sync_copy(dvm, dis_hbm.at[pl.ds(s * 640, 640)])


@functools.partial(
    pl.kernel,
    out_type=jax.ShapeDtypeStruct((32, EB), jnp.float32),
    mesh=_mesh,
    compiler_params=_CP,
    scratch_types=[
        pltpu.VMEM((EB,), jnp.int32),
        pltpu.VMEM((EB,), jnp.int32),
        pltpu.VMEM((EB,), jnp.float32),
        pltpu.VMEM((NPAD4,), jnp.float32),
        pltpu.VMEM((EB,), jnp.float32),
    ],
)
def _norm_kernel(row_hbm, col_hbm, w_hbm, dis_hbm, norm_hbm,
                 rv, cv, wv, disv, nv):
    c = lax.axis_index("c")
    s = lax.axis_index("s")
    wid = s * NCORES + c
    pltpu.sync_copy(row_hbm.at[wid], rv)
    pltpu.sync_copy(col_hbm.at[wid], cv)
    pltpu.sync_copy(w_hbm.at[wid], wv)
    pltpu.sync_copy(dis_hbm, disv)

    def body(g, carry):
        off = g * 16
        r16 = rv[pl.ds(off, 16)]
        c16 = cv[pl.ds(off, 16)]
        w16 = wv[pl.ds(off, 16)]
        dr = plsc.load_gather(disv, [r16])
        dc = plsc.load_gather(disv, [c16])
        nv[pl.ds(off, 16)] = dr * w16 * dc
        return carry

    lax.fori_loop(0, GB, body, 0)
    pltpu.sync_copy(nv, norm_hbm.at[wid])


@functools.partial(
    pl.kernel,
    out_type=(
        jax.ShapeDtypeStruct((NR, 32, SLOTR * 16), jnp.int32),
        jax.ShapeDtypeStruct((NR, 32, SLOTR * 16), jnp.int32),
        jax.ShapeDtypeStruct((NR, 32, SLOTR * 16), jnp.float32),
        jax.ShapeDtypeStruct((NR, 32, 16), jnp.int32),
    ),
    mesh=_mesh,
    compiler_params=_CP,
    scratch_types=[
        pltpu.VMEM((EB,), jnp.int32),
        pltpu.VMEM((EB,), jnp.int32),
        pltpu.VMEM((EB,), jnp.float32),
        pltpu.VMEM((SLOTR * 16,), jnp.int32),
        pltpu.VMEM((SLOTR * 16,), jnp.int32),
        pltpu.VMEM((SLOTR * 16,), jnp.float32),
        pltpu.VMEM((16,), jnp.int32),
    ],
)
def _compact_kernel(row_hbm, col_hbm, norm_hbm,
                    lrow_hbm, lcol_hbm, lnorm_hbm, lcnt_hbm,
                    cr, cc, cn, obr, obc, obn, cntv):
    c = lax.axis_index("c")
    s = lax.axis_index("s")
    iota = lax.iota(jnp.int32, 16)
    zi = jnp.zeros((16,), jnp.int32)
    zf = jnp.zeros((16,), jnp.float32)

    for p in range(2):
        rg = (s * NCORES + c) * 2 + p
        lo = rg * RR
        hi = lo + RR

        def slot(ch, carry0):
            pltpu.sync_copy(row_hbm.at[ch], cr)
            pltpu.sync_copy(col_hbm.at[ch], cc)
            pltpu.sync_copy(norm_hbm.at[ch], cn)

            def body(g, fill):
                off = g * 16
                r16 = cr[pl.ds(off, 16)]
                c16 = cc[pl.ds(off, 16)]
                n16 = cn[pl.ds(off, 16)]
                m = (c16 >= lo) & (c16 < hi)
                incl = plsc.cumsum(jnp.where(m, 1, 0))
                pos = fill + incl - 1
                plsc.store_scatter(obr, [pos], r16, mask=m)
                plsc.store_scatter(obc, [pos], c16 - lo, mask=m)
                plsc.store_scatter(obn, [pos], n16, mask=m)
                return fill + plsc.all_reduce_population_count(m)

            fill = lax.fori_loop(0, GB, body, zi)
            pad = jnp.bitwise_and(16 - jnp.bitwise_and(fill, 15), 15)
            m = iota < pad
            pos = fill + plsc.cumsum(jnp.where(m, 1, 0)) - 1
            plsc.store_scatter(obr, [pos], zi, mask=m)
            plsc.store_scatter(obc, [pos], zi, mask=m)
            plsc.store_scatter(obn, [pos], zf, mask=m)
            fillp = fill + plsc.all_reduce_population_count(m)
            pltpu.sync_copy(obr, lrow_hbm.at[rg, ch])
            pltpu.sync_copy(obc, lcol_hbm.at[rg, ch])
            pltpu.sync_copy(obn, lnorm_hbm.at[rg, ch])
            cntv[...] = lax.shift_right_logical(fillp, 4)
            pltpu.sync_copy(cntv, lcnt_hbm.at[rg, ch])
            return carry0

        lax.fori_loop(0, 32, slot, 0)


def _narrow_hops(h0_hbm, inits, outs, row_hbm, col_hbm, norm_hbm,
                 hv, msg, ivh, rv, cv, nvv, acc, s, write_last_only):
    pltpu.sync_copy(h0_hbm, hv)

    for k in range(3):
        pltpu.sync_copy(inits[k].at[pl.ds(s * 2560, 2560)],
                        acc.at[pl.ds(s * 2560, 2560)])
        plsc.subcore_barrier()
        for h in range(2):
            pltpu.sync_copy(row_hbm.at[s, h], rv)
            pltpu.sync_copy(col_hbm.at[s, h], cv)
            pltpu.sync_copy(norm_hbm.at[s, h], nvv)

            def body(g, carry):
                off = g * 16
                r16 = rv[pl.ds(off, 16)]
                c16 = cv[pl.ds(off, 16)]
                n16 = nvv[pl.ds(off, 16)]
                r4 = r16 * 4
                c4 = c16 * 4
                e4 = (off + lax.iota(jnp.int32, 16)) * 4
                for f in range(4):
                    vals = plsc.load_gather(hv, [r4 + f]) * n16
                    plsc.store_scatter(msg, [e4 + f], vals)
                    plsc.store_scatter(ivh, [e4 + f], c4 + f)
                return carry

            lax.fori_loop(0, GCH, body, 0, unroll=2)
            pltpu.sync_copy(msg, acc.at[ivh], add=True)
        plsc.subcore_barrier()
        if k < 2:
            pltpu.sync_copy(acc, hv)
        if (not write_last_only) or k == 2:
            pltpu.sync_copy(acc.at[pl.ds(s * 2560, 2560)],
                            outs[k].at[pl.ds(s * 2560, 2560)])
        plsc.subcore_barrier()


_NARROW_SCRATCH = [
    pltpu.VMEM((NPAD4 * 4,), jnp.float32),
    pltpu.VMEM((MCH,), jnp.float32),
    pltpu.VMEM((MCH,), jnp.int32),
    pltpu.VMEM((ECH,), jnp.int32),
    pltpu.VMEM((ECH,), jnp.int32),
    pltpu.VMEM((ECH,), jnp.float32),
    pltpu.VMEM_SHARED((NPAD4 * 4,), jnp.float32),
]


@functools.partial(
    pl.kernel,
    out_type=tuple(jax.ShapeDtypeStruct((NPAD4 * 4,), jnp.float32)
                   for _ in range(3)),
    mesh=_mesh,
    compiler_params=_CP,
    scratch_types=_NARROW_SCRATCH,
)
def _layer1_prop_kernel(x_hbm, zeros_hbm, row_hbm, col_hbm, norm_hbm,
                        p1, p2, p3, hv, msg, ivh, rv, cv, nvv, acc):
    c = lax.axis_index("c")
    s = lax.axis_index("s")

    @pl.when(c == 0)
    def _():
        _narrow_hops(x_hbm, [zeros_hbm] * 3, [p1, p2, p3],
                     row_hbm, col_hbm, norm_hbm,
                     hv, msg, ivh, rv, cv, nvv, acc, s, False)


@functools.partial(
    pl.kernel,
    out_type=jax.ShapeDtypeStruct((NPAD4 * 4,), jnp.float32),
    mesh=_mesh,
    compiler_params=_CP,
    scratch_types=_NARROW_SCRATCH,
)
def _layer3_prop_kernel(t0, t1, t2, t3, row_hbm, col_hbm, norm_hbm,
                        out, hv, msg, ivh, rv, cv, nvv, acc):
    c = lax.axis_index("c")
    s = lax.axis_index("s")

    @pl.when(c == 0)
    def _():
        _narrow_hops(t3, [t2, t1, t0], [out, out, out],
                     row_hbm, col_hbm, norm_hbm,
                     hv, msg, ivh, rv, cv, nvv, acc, s, True)


@functools.partial(
    pl.kernel,
    out_type=jax.ShapeDtypeStruct((NR * RR * H,), jnp.float32),
    mesh=_mesh,
    compiler_params=_CP,
    scratch_types=[
        pltpu.VMEM((1024,), jnp.int32),
        pltpu.VMEM((1024,), jnp.int32),
        pltpu.VMEM((1024,), jnp.float32),
        pltpu.VMEM((16, H), jnp.float32),
        pltpu.VMEM((16, H), jnp.float32),
        pltpu.VMEM((16,), jnp.int32),
        pltpu.VMEM((ACC1,), jnp.float32),
        pltpu.SemaphoreType.DMA,
        pltpu.SemaphoreType.DMA,
    ],
)
def _wide_prop_kernel(h_hbm, lrow_hbm, lcol_hbm, lnorm_hbm, lcnt_hbm,
                      zeros_hbm, out_hbm,
                      cbr, cbc, cbn, rows0, rows1, cntv, acc1, sem0, sem1):
    c = lax.axis_index("c")
    s = lax.axis_index("s")
    iota = lax.iota(jnp.int32, 16)

    def start(gv, buf, sem_):
        pltpu.async_copy(h_hbm.at[cbr.at[pl.ds(gv * 16, 16)]], buf, sem_)

    def waitbuf(buf, sem_):
        pltpu.make_async_copy(h_hbm.at[pl.ds(0, 16)], buf, sem_).wait()

    def do_group(gv, buf):
        off = gv * 16
        cl16 = cbc[pl.ds(off, 16)]
        n16 = cbn[pl.ds(off, 16)]
        for e in range(16):
            nb = _blane(n16, e)
            base = jnp.max(jnp.where(iota == e, cl16, 0)) * H
            for j in range(H // 16):
                sl = pl.ds(base + j * 16, 16)
                acc1[sl] = acc1[sl] + buf[e, pl.ds(j * 16, 16)] * nb

    def prange(p, carryp):
        rg = (s * NCORES + c) * 2 + p
        pltpu.sync_copy(zeros_hbm, acc1)

        def slot(ch, carry0):
            pltpu.sync_copy(lcnt_hbm.at[rg, ch], cntv)
            trips = jnp.max(cntv[...])
            nco = lax.shift_right_logical(trips + 63, 6)

            def chunk(co, carry):
                base = co * 64
                pltpu.sync_copy(lrow_hbm.at[rg, ch, pl.ds(base * 16, 1024)], cbr)
                pltpu.sync_copy(lcol_hbm.at[rg, ch, pl.ds(base * 16, 1024)], cbc)
                pltpu.sync_copy(lnorm_hbm.at[rg, ch, pl.ds(base * 16, 1024)], cbn)
                nin = jnp.minimum(64, trips - base)

                @pl.when(nin > 0)
                def _():
                    start(0, rows0, sem0)

                @pl.when(nin > 1)
                def _():
                    start(1, rows1, sem1)

                def body2(i, carry2):
                    g0 = 2 * i
                    g1 = g0 + 1
                    waitbuf(rows0, sem0)
                    do_group(g0, rows0)

                    @pl.when(g0 + 2 < nin)
                    def _():
                        start(g0 + 2, rows0, sem0)

                    @pl.when(g1 < nin)
                    def _():
                        waitbuf(rows1, sem1)
                        do_group(g1, rows1)

                        @pl.when(g1 + 2 < nin)
                        def __():
                            start(g1 + 2, rows1, sem1)

                    return carry2

                lax.fori_loop(0, lax.shift_right_logical(nin + 1, 1), body2, 0)
                return carry

            lax.fori_loop(0, nco, chunk, 0)
            return carry0

        lax.fori_loop(0, 32, slot, 0)
        pltpu.sync_copy(acc1, out_hbm.at[pl.ds(rg * ACC1, ACC1)])
        return carryp

    lax.fori_loop(0, 2, prange, 0)


_PREC = lax.Precision.HIGHEST


def _elu(x):
    return jnp.where(x > 0.0, x, jnp.exp(jnp.minimum(x, 0.0)) - 1.0)


def _tc_lin_stats(xcat, w, b):
    n, fin = xcat.shape

    def body(x_ref, w_ref, b_ref, out_ref, st_ref):
        i = pl.program_id(0)
        h = jnp.dot(x_ref[...], w_ref[...],
                    preferred_element_type=jnp.float32, precision=_PREC)
        h = _elu(h + b_ref[...])
        out_ref[...] = h

        @pl.when(i == 0)
        def _():
            st_ref[...] = jnp.zeros_like(st_ref)

        st_ref[0:1, :] += jnp.sum(h, axis=0, keepdims=True)
        st_ref[1:2, :] += jnp.sum(h * h, axis=0, keepdims=True)

    return pl.pallas_call(
        body,
        grid=(10,),
        in_specs=[
            pl.BlockSpec((n // 10, fin), lambda i: (i, 0)),
            pl.BlockSpec((fin, H), lambda i: (0, 0)),
            pl.BlockSpec((H,), lambda i: (0,)),
        ],
        out_specs=[
            pl.BlockSpec((n // 10, H), lambda i: (i, 0)),
            pl.BlockSpec((8, H), lambda i: (0, 0)),
        ],
        out_shape=[
            jax.ShapeDtypeStruct((n, H), jnp.float32),
            jax.ShapeDtypeStruct((8, H), jnp.float32),
        ],
    )(xcat, w, b)


def _tc_lin4_stats(h1, z1, z2, z3, w2, b):
    n = h1.shape[0]

    def body(a_ref, b_ref_, c_ref, d_ref, w_ref, bias_ref, out_ref, st_ref):
        i = pl.program_id(0)
        h = jnp.dot(a_ref[...], w_ref[0],
                    preferred_element_type=jnp.float32, precision=_PREC)
        h += jnp.dot(b_ref_[...], w_ref[1],
                     preferred_element_type=jnp.float32, precision=_PREC)
        h += jnp.dot(c_ref[...], w_ref[2],
                     preferred_element_type=jnp.float32, precision=_PREC)
        h += jnp.dot(d_ref[...], w_ref[3],
                     preferred_element_type=jnp.float32, precision=_PREC)
        h = _elu(h + bias_ref[...])
        out_ref[...] = h

        @pl.when(i == 0)
        def _():
            st_ref[...] = jnp.zeros_like(st_ref)

        st_ref[0:1, :] += jnp.sum(h, axis=0, keepdims=True)
        st_ref[1:2, :] += jnp.sum(h * h, axis=0, keepdims=True)

    blk = pl.BlockSpec((n // 10, H), lambda i: (i, 0))
    return pl.pallas_call(
        body,
        grid=(10,),
        in_specs=[blk, blk, blk, blk,
                  pl.BlockSpec((4, H, H), lambda i: (0, 0, 0)),
                  pl.BlockSpec((H,), lambda i: (0,))],
        out_specs=[blk, pl.BlockSpec((8, H), lambda i: (0, 0))],
        out_shape=[
            jax.ShapeDtypeStruct((n, H), jnp.float32),
            jax.ShapeDtypeStruct((8, H), jnp.float32),
        ],
    )(h1, z1, z2, z3, w2, b)


def _tc_graph_norm(hpre, stats, w, b, ms):
    n = hpre.shape[0]

    def body(h_ref, st_ref, w_ref, b_ref, ms_ref, out_ref):
        m = st_ref[0:1, :] * (1.0 / n)
        s2 = st_ref[1:2, :] * (1.0 / n)
        msv = ms_ref[...]
        var = s2 - 2.0 * msv * m * m + msv * msv * m * m
        inv = lax.rsqrt(var + 1e-5)
        out_ref[...] = w_ref[...] * (h_ref[...] - m * msv) * inv + b_ref[...]

    return pl.pallas_call(
        body,
        grid=(10,),
        in_specs=[
            pl.BlockSpec((n // 10, H), lambda i: (i, 0)),
            pl.BlockSpec((8, H), lambda i: (0, 0)),
            pl.BlockSpec((H,), lambda i: (0,)),
            pl.BlockSpec((H,), lambda i: (0,)),
            pl.BlockSpec((H,), lambda i: (0,)),
        ],
        out_specs=pl.BlockSpec((n // 10, H), lambda i: (i, 0)),
        out_shape=jax.ShapeDtypeStruct((n, H), jnp.float32),
    )(hpre, stats, w, b, ms)


def _tc_final_lin(h2, w3r, b3r):
    n = h2.shape[0]

    def body(h_ref, w_ref, b_ref, out_ref):
        out_ref[...] = jnp.dot(
            h_ref[...], w_ref[...],
            preferred_element_type=jnp.float32, precision=_PREC) + b_ref[...]

    return pl.pallas_call(
        body,
        grid=(10,),
        in_specs=[
            pl.BlockSpec((n // 10, H), lambda i: (i, 0)),
            pl.BlockSpec((H, 16), lambda i: (0, 0)),
            pl.BlockSpec((16,), lambda i: (0,)),
        ],
        out_specs=pl.BlockSpec((n // 10, 16), lambda i: (i, 0)),
        out_shape=jax.ShapeDtypeStruct((n, 16), jnp.float32),
    )(h2, w3r, b3r)


def kernel(x, edge_index, weight, W1, b1, W2, b2, W3, b3,
           gn1_w, gn1_b, gn1_ms, gn2_w, gn2_b, gn2_ms):
    i32 = jnp.int32
    f32 = jnp.float32
    pad_e = EPAD - E
    rowp = jnp.concatenate([edge_index[0], jnp.zeros((pad_e,), i32)])
    colp = jnp.concatenate([edge_index[1], jnp.zeros((pad_e,), i32)])
    wp = jnp.concatenate([weight, jnp.zeros((pad_e,), f32)])

    zeros1 = jnp.zeros((NPAD4,), f32)
    zeros4 = jnp.zeros((NPAD4 * 4,), f32)
    zerosw = jnp.zeros((ACC1,), f32)

    dis = _deg_dis_kernel(colp.reshape(NT, EN), wp.reshape(NT, EN), zeros1)
    norm2 = _norm_kernel(rowp.reshape(32, EB), colp.reshape(32, EB),
                         wp.reshape(32, EB), dis)

    lrow, lcol, lnorm, lcnt = _compact_kernel(
        rowp.reshape(32, EB), colp.reshape(32, EB), norm2)

    rowN = rowp.reshape(NT, 2, ECH)
    colN = colp.reshape(NT, 2, ECH)
    normN = norm2.reshape(NT, 2, ECH)

    xflat = jnp.pad(x, ((0, NPAD4 - N), (0, 0))).reshape(-1)
    p1, p2, p3 = _layer1_prop_kernel(xflat, zeros4, rowN, colN, normN)

    xcat = jnp.concatenate(
        [x, p1.reshape(NPAD4, 4)[:N], p2.reshape(NPAD4, 4)[:N],
         p3.reshape(NPAD4, 4)[:N]], axis=1)
    h1pre, st1 = _tc_lin_stats(xcat, W1.reshape(4 * F_IN, H), b1)
    h1 = _tc_graph_norm(h1pre, st1, gn1_w, gn1_b, gn1_ms)

    h1p = jnp.pad(h1, ((0, NPADW - N), (0, 0)))
    z1r = _wide_prop_kernel(h1p, lrow, lcol, lnorm, lcnt,
                            zerosw).reshape(NPADW, H)
    z2r = _wide_prop_kernel(z1r, lrow, lcol, lnorm, lcnt,
                            zerosw).reshape(NPADW, H)
    z3r = _wide_prop_kernel(z2r, lrow, lcol, lnorm, lcnt,
                            zerosw).reshape(NPADW, H)

    h2pre, st2 = _tc_lin4_stats(h1, z1r[:N], z2r[:N], z3r[:N], W2, b2)
    h2 = _tc_graph_norm(h2pre, st2, gn2_w, gn2_b, gn2_ms)

    w3r = jnp.transpose(W3, (1, 0, 2)).reshape(H, 16)
    b3r = jnp.concatenate([b3, jnp.zeros((12,), f32)])
    t = _tc_final_lin(h2, w3r, b3r)
    tp = jnp.pad(t, ((0, NPAD4 - N), (0, 0)))
    t0 = tp[:, 0:4].reshape(-1)
    t1 = tp[:, 4:8].reshape(-1)
    t2 = tp[:, 8:12].reshape(-1)
    t3 = tp[:, 12:16].reshape(-1)

    out = _layer3_prop_kernel(t0, t1, t2, t3, rowN, colN, normN)
    return out.reshape(NPAD4, 4)[:N]


# speedup vs baseline: 1.2265x; 1.2265x over previous
import functools

import jax
import jax.numpy as jnp
from jax import lax
from jax.experimental import pallas as pl
from jax.experimental.pallas import tpu as pltpu
from jax.experimental.pallas import tpu_sc as plsc

N = 10000
E = 160000
F_IN = 4
H = 512

NT = 16
NCORES = 2

NPAD4 = 10240
EPAD = 160256
EN = EPAD // NT
ECH = EN // 2
GCH = ECH // 16
MCH = ECH * 4
EB = EPAD // 32
GB = EB // 16

RR = 157
NR = 64
NPADW = RR * NR
ACC1 = RR * H
SLOTR = 320

_mesh = plsc.VectorSubcoreMesh(core_axis_name="c", subcore_axis_name="s")
_CP = pltpu.CompilerParams(needs_layout_passes=False)


_GDN = lax.GatherDimensionNumbers(
    offset_dims=(), collapsed_slice_dims=(0,), start_index_map=(0,))


def _blane(v, e):
    idx = jnp.full((16, 1), e, jnp.int32)
    return lax.gather(v, idx, _GDN, (1,),
                      mode=lax.GatherScatterMode.PROMISE_IN_BOUNDS)


def _rsqrt16(x):
    i = lax.bitcast_convert_type(x, jnp.int32)
    i = jnp.int32(0x5F3759DF) - lax.shift_right_logical(i, 1)
    y = lax.bitcast_convert_type(i, jnp.float32)
    for _ in range(3):
        y = y * (1.5 - 0.5 * x * y * y)
    return y


@functools.partial(
    pl.kernel,
    out_type=jax.ShapeDtypeStruct((NPAD4,), jnp.float32),
    mesh=_mesh,
    compiler_params=_CP,
    scratch_types=[
        pltpu.VMEM((EN,), jnp.int32),
        pltpu.VMEM((EN,), jnp.float32),
        pltpu.VMEM((640,), jnp.float32),
        pltpu.VMEM_SHARED((NPAD4,), jnp.float32),
    ],
)
def _deg_dis_kernel(col_hbm, w_hbm, zeros_hbm, dis_hbm, colv, wv, dvm, deg):
    c = lax.axis_index("c")
    s = lax.axis_index("s")

    @pl.when(c == 0)
    def _():
        pltpu.sync_copy(col_hbm.at[s], colv)
        pltpu.sync_copy(w_hbm.at[s], wv)
        pltpu.sync_copy(zeros_hbm.at[pl.ds(s * 640, 640)],
                        deg.at[pl.ds(s * 640, 640)])
        plsc.subcore_barrier()
        pltpu.sync_copy(wv, deg.at[colv], add=True)
        plsc.subcore_barrier()
        pltpu.sync_copy(deg.at[pl.ds(s * 640, 640)], dvm)

        def body(g, carry):
            d16 = dvm[pl.ds(g * 16, 16)]
            y = _rsqrt16(d16)
            dvm[pl.ds(g * 16, 16)] = jnp.where(d16 > 0.0, y, 0.0)
            return carry

        lax.fori_loop(0, 40, body, 0)
        pltpu.sync_copy(dvm, dis_hbm.at[pl.ds(s * 640, 640)])


@functools.partial(
    pl.kernel,
    out_type=jax.ShapeDtypeStruct((32, EB), jnp.float32),
    mesh=_mesh,
    compiler_params=_CP,
    scratch_types=[
        pltpu.VMEM((EB,), jnp.int32),
        pltpu.VMEM((EB,), jnp.int32),
        pltpu.VMEM((EB,), jnp.float32),
        pltpu.VMEM((NPAD4,), jnp.float32),
        pltpu.VMEM((EB,), jnp.float32),
    ],
)
def _norm_kernel(row_hbm, col_hbm, w_hbm, dis_hbm, norm_hbm,
                 rv, cv, wv, disv, nv):
    c = lax.axis_index("c")
    s = lax.axis_index("s")
    wid = s * NCORES + c
    pltpu.sync_copy(row_hbm.at[wid], rv)
    pltpu.sync_copy(col_hbm.at[wid], cv)
    pltpu.sync_copy(w_hbm.at[wid], wv)
    pltpu.sync_copy(dis_hbm, disv)

    def body(g, carry):
        off = g * 16
        r16 = rv[pl.ds(off, 16)]
        c16 = cv[pl.ds(off, 16)]
        w16 = wv[pl.ds(off, 16)]
        dr = plsc.load_gather(disv, [r16])
        dc = plsc.load_gather(disv, [c16])
        nv[pl.ds(off, 16)] = dr * w16 * dc
        return carry

    lax.fori_loop(0, GB, body, 0)
    pltpu.sync_copy(nv, norm_hbm.at[wid])


@functools.partial(
    pl.kernel,
    out_type=(
        jax.ShapeDtypeStruct((NR, 32, SLOTR * 16), jnp.int32),
        jax.ShapeDtypeStruct((NR, 32, SLOTR * 16), jnp.int32),
        jax.ShapeDtypeStruct((NR, 32, SLOTR * 16), jnp.float32),
        jax.ShapeDtypeStruct((NR, 32, 16), jnp.int32),
    ),
    mesh=_mesh,
    compiler_params=_CP,
    scratch_types=[
        pltpu.VMEM((EB,), jnp.int32),
        pltpu.VMEM((EB,), jnp.int32),
        pltpu.VMEM((EB,), jnp.float32),
        pltpu.VMEM((SLOTR * 16,), jnp.int32),
        pltpu.VMEM((SLOTR * 16,), jnp.int32),
        pltpu.VMEM((SLOTR * 16,), jnp.float32),
        pltpu.VMEM((16,), jnp.int32),
    ],
)
def _compact_kernel(row_hbm, col_hbm, norm_hbm,
                    lrow_hbm, lcol_hbm, lnorm_hbm, lcnt_hbm,
                    cr, cc, cn, obr, obc, obn, cntv):
    c = lax.axis_index("c")
    s = lax.axis_index("s")
    iota = lax.iota(jnp.int32, 16)
    zi = jnp.zeros((16,), jnp.int32)
    zf = jnp.zeros((16,), jnp.float32)

    for p in range(2):
        rg = (s * NCORES + c) * 2 + p
        lo = rg * RR
        hi = lo + RR

        def slot(ch, carry0):
            pltpu.sync_copy(row_hbm.at[ch], cr)
            pltpu.sync_copy(col_hbm.at[ch], cc)
            pltpu.sync_copy(norm_hbm.at[ch], cn)

            def body(g, fill):
                off = g * 16
                r16 = cr[pl.ds(off, 16)]
                c16 = cc[pl.ds(off, 16)]
                n16 = cn[pl.ds(off, 16)]
                m = (c16 >= lo) & (c16 < hi)
                incl = plsc.cumsum(jnp.where(m, 1, 0))
                pos = fill + incl - 1
                plsc.store_scatter(obr, [pos], r16, mask=m)
                plsc.store_scatter(obc, [pos], c16 - lo, mask=m)
                plsc.store_scatter(obn, [pos], n16, mask=m)
                return fill + plsc.all_reduce_population_count(m)

            fill = lax.fori_loop(0, GB, body, zi)
            pad = jnp.bitwise_and(16 - jnp.bitwise_and(fill, 15), 15)
            m = iota < pad
            pos = fill + plsc.cumsum(jnp.where(m, 1, 0)) - 1
            plsc.store_scatter(obr, [pos], zi, mask=m)
            plsc.store_scatter(obc, [pos], zi, mask=m)
            plsc.store_scatter(obn, [pos], zf, mask=m)
            fillp = fill + plsc.all_reduce_population_count(m)
            pltpu.sync_copy(obr, lrow_hbm.at[rg, ch])
            pltpu.sync_copy(obc, lcol_hbm.at[rg, ch])
            pltpu.sync_copy(obn, lnorm_hbm.at[rg, ch])
            cntv[...] = lax.shift_right_logical(fillp, 4)
            pltpu.sync_copy(cntv, lcnt_hbm.at[rg, ch])
            return carry0

        lax.fori_loop(0, 32, slot, 0)


def _narrow_hops(h0_hbm, inits, outs, row_hbm, col_hbm, norm_hbm,
                 hv, msg, ivh, rv, cv, nvv, acc, s, write_last_only):
    pltpu.sync_copy(h0_hbm, hv)

    for k in range(3):
        pltpu.sync_copy(inits[k].at[pl.ds(s * 2560, 2560)],
                        acc.at[pl.ds(s * 2560, 2560)])
        plsc.subcore_barrier()
        for h in range(2):
            pltpu.sync_copy(row_hbm.at[s, h], rv)
            pltpu.sync_copy(col_hbm.at[s, h], cv)
            pltpu.sync_copy(norm_hbm.at[s, h], nvv)

            def body(g, carry):
                off = g * 16
                r16 = rv[pl.ds(off, 16)]
                c16 = cv[pl.ds(off, 16)]
                n16 = nvv[pl.ds(off, 16)]
                r4 = r16 * 4
                c4 = c16 * 4
                e4 = (off + lax.iota(jnp.int32, 16)) * 4
                for f in range(4):
                    vals = plsc.load_gather(hv, [r4 + f]) * n16
                    plsc.store_scatter(msg, [e4 + f], vals)
                    plsc.store_scatter(ivh, [e4 + f], c4 + f)
                return carry

            lax.fori_loop(0, GCH, body, 0, unroll=2)
            pltpu.sync_copy(msg, acc.at[ivh], add=True)
        plsc.subcore_barrier()
        if k < 2:
            pltpu.sync_copy(acc, hv)
        if (not write_last_only) or k == 2:
            pltpu.sync_copy(acc.at[pl.ds(s * 2560, 2560)],
                            outs[k].at[pl.ds(s * 2560, 2560)])
        plsc.subcore_barrier()


_NARROW_SCRATCH = [
    pltpu.VMEM((NPAD4 * 4,), jnp.float32),
    pltpu.VMEM((MCH,), jnp.float32),
    pltpu.VMEM((MCH,), jnp.int32),
    pltpu.VMEM((ECH,), jnp.int32),
    pltpu.VMEM((ECH,), jnp.int32),
    pltpu.VMEM((ECH,), jnp.float32),
    pltpu.VMEM_SHARED((NPAD4 * 4,), jnp.float32),
]


@functools.partial(
    pl.kernel,
    out_type=tuple(jax.ShapeDtypeStruct((NPAD4 * 4,), jnp.float32)
                   for _ in range(3)),
    mesh=_mesh,
    compiler_params=_CP,
    scratch_types=_NARROW_SCRATCH,
)
def _layer1_prop_kernel(x_hbm, zeros_hbm, row_hbm, col_hbm, norm_hbm,
                        p1, p2, p3, hv, msg, ivh, rv, cv, nvv, acc):
    c = lax.axis_index("c")
    s = lax.axis_index("s")

    @pl.when(c == 0)
    def _():
        _narrow_hops(x_hbm, [zeros_hbm] * 3, [p1, p2, p3],
                     row_hbm, col_hbm, norm_hbm,
                     hv, msg, ivh, rv, cv, nvv, acc, s, False)


@functools.partial(
    pl.kernel,
    out_type=jax.ShapeDtypeStruct((NPAD4 * 4,), jnp.float32),
    mesh=_mesh,
    compiler_params=_CP,
    scratch_types=_NARROW_SCRATCH,
)
def _layer3_prop_kernel(t0, t1, t2, t3, row_hbm, col_hbm, norm_hbm,
                        out, hv, msg, ivh, rv, cv, nvv, acc):
    c = lax.axis_index("c")
    s = lax.axis_index("s")

    @pl.when(c == 0)
    def _():
        _narrow_hops(t3, [t2, t1, t0], [out, out, out],
                     row_hbm, col_hbm, norm_hbm,
                     hv, msg, ivh, rv, cv, nvv, acc, s, True)


@functools.partial(
    pl.kernel,
    out_type=jax.ShapeDtypeStruct((NR * RR * H,), jnp.float32),
    mesh=_mesh,
    compiler_params=_CP,
    scratch_types=[
        pltpu.VMEM((1024,), jnp.int32),
        pltpu.VMEM((1024,), jnp.int32),
        pltpu.VMEM((1024,), jnp.float32),
        pltpu.VMEM((16, H), jnp.float32),
        pltpu.VMEM((16, H), jnp.float32),
        pltpu.VMEM((16,), jnp.int32),
        pltpu.VMEM((ACC1,), jnp.float32),
        pltpu.SemaphoreType.DMA,
        pltpu.SemaphoreType.DMA,
    ],
)
def _wide_prop_kernel(h_hbm, lrow_hbm, lcol_hbm, lnorm_hbm, lcnt_hbm,
                      zeros_hbm, out_hbm,
                      cbr, cbc, cbn, rows0, rows1, cntv, acc1, sem0, sem1):
    c = lax.axis_index("c")
    s = lax.axis_index("s")
    iota = lax.iota(jnp.int32, 16)

    def start(gv, buf, sem_):
        pltpu.async_copy(h_hbm.at[cbr.at[pl.ds(gv * 16, 16)]], buf, sem_)

    def waitbuf(buf, sem_):
        pltpu.make_async_copy(h_hbm.at[pl.ds(0, 16)], buf, sem_).wait()

    def do_group(gv, buf):
        off = gv * 16
        cl16 = cbc[pl.ds(off, 16)]
        n16 = cbn[pl.ds(off, 16)]
        for e in range(16):
            nb = _blane(n16, e)
            bidx = _blane(cl16, e) * H + iota
            for j in range(H // 16):
                plsc.addupdate_scatter(
                    acc1, [bidx + j * 16], buf[e, pl.ds(j * 16, 16)] * nb)

    def prange(p, carryp):
        rg = (s * NCORES + c) * 2 + p
        pltpu.sync_copy(zeros_hbm, acc1)

        def slot(ch, carry0):
            pltpu.sync_copy(lcnt_hbm.at[rg, ch], cntv)
            trips = jnp.max(cntv[...])
            nco = lax.shift_right_logical(trips + 63, 6)

            def chunk(co, carry):
                base = co * 64
                pltpu.sync_copy(lrow_hbm.at[rg, ch, pl.ds(base * 16, 1024)], cbr)
                pltpu.sync_copy(lcol_hbm.at[rg, ch, pl.ds(base * 16, 1024)], cbc)
                pltpu.sync_copy(lnorm_hbm.at[rg, ch, pl.ds(base * 16, 1024)], cbn)
                nin = jnp.minimum(64, trips - base)

                @pl.when(nin > 0)
                def _():
                    start(0, rows0, sem0)

                @pl.when(nin > 1)
                def _():
                    start(1, rows1, sem1)

                def body2(i, carry2):
                    g0 = 2 * i
                    g1 = g0 + 1
                    waitbuf(rows0, sem0)
                    do_group(g0, rows0)

                    @pl.when(g0 + 2 < nin)
                    def _():
                        start(g0 + 2, rows0, sem0)

                    @pl.when(g1 < nin)
                    def _():
                        waitbuf(rows1, sem1)
                        do_group(g1, rows1)

                        @pl.when(g1 + 2 < nin)
                        def __():
                            start(g1 + 2, rows1, sem1)

                    return carry2

                lax.fori_loop(0, lax.shift_right_logical(nin + 1, 1), body2, 0)
                return carry

            lax.fori_loop(0, nco, chunk, 0)
            return carry0

        lax.fori_loop(0, 32, slot, 0)
        pltpu.sync_copy(acc1, out_hbm.at[pl.ds(rg * ACC1, ACC1)])
        return carryp

    lax.fori_loop(0, 2, prange, 0)


_PREC = lax.Precision.HIGHEST


def _elu(x):
    return jnp.where(x > 0.0, x, jnp.exp(jnp.minimum(x, 0.0)) - 1.0)


def _tc_lin_stats(xcat, w, b):
    n, fin = xcat.shape

    def body(x_ref, w_ref, b_ref, out_ref, st_ref):
        i = pl.program_id(0)
        h = jnp.dot(x_ref[...], w_ref[...],
                    preferred_element_type=jnp.float32, precision=_PREC)
        h = _elu(h + b_ref[...])
        out_ref[...] = h

        @pl.when(i == 0)
        def _():
            st_ref[...] = jnp.zeros_like(st_ref)

        st_ref[0:1, :] += jnp.sum(h, axis=0, keepdims=True)
        st_ref[1:2, :] += jnp.sum(h * h, axis=0, keepdims=True)

    return pl.pallas_call(
        body,
        grid=(10,),
        in_specs=[
            pl.BlockSpec((n // 10, fin), lambda i: (i, 0)),
            pl.BlockSpec((fin, H), lambda i: (0, 0)),
            pl.BlockSpec((H,), lambda i: (0,)),
        ],
        out_specs=[
            pl.BlockSpec((n // 10, H), lambda i: (i, 0)),
            pl.BlockSpec((8, H), lambda i: (0, 0)),
        ],
        out_shape=[
            jax.ShapeDtypeStruct((n, H), jnp.float32),
            jax.ShapeDtypeStruct((8, H), jnp.float32),
        ],
    )(xcat, w, b)


def _tc_lin4_stats(h1, z1, z2, z3, w2, b):
    n = h1.shape[0]

    def body(a_ref, b_ref_, c_ref, d_ref, w_ref, bias_ref, out_ref, st_ref):
        i = pl.program_id(0)
        h = jnp.dot(a_ref[...], w_ref[0],
                    preferred_element_type=jnp.float32, precision=_PREC)
        h += jnp.dot(b_ref_[...], w_ref[1],
                     preferred_element_type=jnp.float32, precision=_PREC)
        h += jnp.dot(c_ref[...], w_ref[2],
                     preferred_element_type=jnp.float32, precision=_PREC)
        h += jnp.dot(d_ref[...], w_ref[3],
                     preferred_element_type=jnp.float32, precision=_PREC)
        h = _elu(h + bias_ref[...])
        out_ref[...] = h

        @pl.when(i == 0)
        def _():
            st_ref[...] = jnp.zeros_like(st_ref)

        st_ref[0:1, :] += jnp.sum(h, axis=0, keepdims=True)
        st_ref[1:2, :] += jnp.sum(h * h, axis=0, keepdims=True)

    blk = pl.BlockSpec((n // 10, H), lambda i: (i, 0))
    return pl.pallas_call(
        body,
        grid=(10,),
        in_specs=[blk, blk, blk, blk,
                  pl.BlockSpec((4, H, H), lambda i: (0, 0, 0)),
                  pl.BlockSpec((H,), lambda i: (0,))],
        out_specs=[blk, pl.BlockSpec((8, H), lambda i: (0, 0))],
        out_shape=[
            jax.ShapeDtypeStruct((n, H), jnp.float32),
            jax.ShapeDtypeStruct((8, H), jnp.float32),
        ],
    )(h1, z1, z2, z3, w2, b)


def _tc_graph_norm(hpre, stats, w, b, ms):
    n = hpre.shape[0]

    def body(h_ref, st_ref, w_ref, b_ref, ms_ref, out_ref):
        m = st_ref[0:1, :] * (1.0 / n)
        s2 = st_ref[1:2, :] * (1.0 / n)
        msv = ms_ref[...]
        var = s2 - 2.0 * msv * m * m + msv * msv * m * m
        inv = lax.rsqrt(var + 1e-5)
        out_ref[...] = w_ref[...] * (h_ref[...] - m * msv) * inv + b_ref[...]

    return pl.pallas_call(
        body,
        grid=(10,),
        in_specs=[
            pl.BlockSpec((n // 10, H), lambda i: (i, 0)),
            pl.BlockSpec((8, H), lambda i: (0, 0)),
            pl.BlockSpec((H,), lambda i: (0,)),
            pl.BlockSpec((H,), lambda i: (0,)),
            pl.BlockSpec((H,), lambda i: (0,)),
        ],
        out_specs=pl.BlockSpec((n // 10, H), lambda i: (i, 0)),
        out_shape=jax.ShapeDtypeStruct((n, H), jnp.float32),
    )(hpre, stats, w, b, ms)


def _tc_final_lin(h2, w3r, b3r):
    n = h2.shape[0]

    def body(h_ref, w_ref, b_ref, out_ref):
        out_ref[...] = jnp.dot(
            h_ref[...], w_ref[...],
            preferred_element_type=jnp.float32, precision=_PREC) + b_ref[...]

    return pl.pallas_call(
        body,
        grid=(10,),
        in_specs=[
            pl.BlockSpec((n // 10, H), lambda i: (i, 0)),
            pl.BlockSpec((H, 16), lambda i: (0, 0)),
            pl.BlockSpec((16,), lambda i: (0,)),
        ],
        out_specs=pl.BlockSpec((n // 10, 16), lambda i: (i, 0)),
        out_shape=jax.ShapeDtypeStruct((n, 16), jnp.float32),
    )(h2, w3r, b3r)


def kernel(x, edge_index, weight, W1, b1, W2, b2, W3, b3,
           gn1_w, gn1_b, gn1_ms, gn2_w, gn2_b, gn2_ms):
    i32 = jnp.int32
    f32 = jnp.float32
    pad_e = EPAD - E
    rowp = jnp.concatenate([edge_index[0], jnp.zeros((pad_e,), i32)])
    colp = jnp.concatenate([edge_index[1], jnp.zeros((pad_e,), i32)])
    wp = jnp.concatenate([weight, jnp.zeros((pad_e,), f32)])

    zeros1 = jnp.zeros((NPAD4,), f32)
    zeros4 = jnp.zeros((NPAD4 * 4,), f32)
    zerosw = jnp.zeros((ACC1,), f32)

    dis = _deg_dis_kernel(colp.reshape(NT, EN), wp.reshape(NT, EN), zeros1)
    norm2 = _norm_kernel(rowp.reshape(32, EB), colp.reshape(32, EB),
                         wp.reshape(32, EB), dis)

    lrow, lcol, lnorm, lcnt = _compact_kernel(
        rowp.reshape(32, EB), colp.reshape(32, EB), norm2)

    rowN = rowp.reshape(NT, 2, ECH)
    colN = colp.reshape(NT, 2, ECH)
    normN = norm2.reshape(NT, 2, ECH)

    xflat = jnp.pad(x, ((0, NPAD4 - N), (0, 0))).reshape(-1)
    p1, p2, p3 = _layer1_prop_kernel(xflat, zeros4, rowN, colN, normN)

    xcat = jnp.concatenate(
        [x, p1.reshape(NPAD4, 4)[:N], p2.reshape(NPAD4, 4)[:N],
         p3.reshape(NPAD4, 4)[:N]], axis=1)
    h1pre, st1 = _tc_lin_stats(xcat, W1.reshape(4 * F_IN, H), b1)
    h1 = _tc_graph_norm(h1pre, st1, gn1_w, gn1_b, gn1_ms)

    h1p = jnp.pad(h1, ((0, NPADW - N), (0, 0)))
    z1r = _wide_prop_kernel(h1p, lrow, lcol, lnorm, lcnt,
                            zerosw).reshape(NPADW, H)
    z2r = _wide_prop_kernel(z1r, lrow, lcol, lnorm, lcnt,
                            zerosw).reshape(NPADW, H)
    z3r = _wide_prop_kernel(z2r, lrow, lcol, lnorm, lcnt,
                            zerosw).reshape(NPADW, H)

    h2pre, st2 = _tc_lin4_stats(h1, z1r[:N], z2r[:N], z3r[:N], W2, b2)
    h2 = _tc_graph_norm(h2pre, st2, gn2_w, gn2_b, gn2_ms)

    w3r = jnp.transpose(W3, (1, 0, 2)).reshape(H, 16)
    b3r = jnp.concatenate([b3, jnp.zeros((12,), f32)])
    t = _tc_final_lin(h2, w3r, b3r)
    tp = jnp.pad(t, ((0, NPAD4 - N), (0, 0)))
    t0 = tp[:, 0:4].reshape(-1)
    t1 = tp[:, 4:8].reshape(-1)
    t2 = tp[:, 8:12].reshape(-1)
    t3 = tp[:, 12:16].reshape(-1)

    out = _layer3_prop_kernel(t0, t1, t2, t3, rowN, colN, normN)
    return out.reshape(NPAD4, 4)[:N]


# speedup vs baseline: 2.1675x; 1.7672x over previous
import functools

import jax
import jax.numpy as jnp
from jax import lax
from jax.experimental import pallas as pl
from jax.experimental.pallas import tpu as pltpu
from jax.experimental.pallas import tpu_sc as plsc

N = 10000
E = 160000
F_IN = 4
H = 512

NT = 16
NCORES = 2

NPAD4 = 10240
EPAD = 160256
EN = EPAD // NT
ECH = EN // 2
GCH = ECH // 16
MCH = ECH * 4
EB = EPAD // 32
GB = EB // 16

RR = 157
NR = 64
NPADW = RR * NR
ACC1 = RR * H
SLOTR = 320

_mesh = plsc.VectorSubcoreMesh(core_axis_name="c", subcore_axis_name="s")
_CP = pltpu.CompilerParams(needs_layout_passes=False)


_GDN = lax.GatherDimensionNumbers(
    offset_dims=(), collapsed_slice_dims=(0,), start_index_map=(0,))


def _blane(v, e):
    idx = jnp.full((16, 1), e, jnp.int32)
    return lax.gather(v, idx, _GDN, (1,),
                      mode=lax.GatherScatterMode.PROMISE_IN_BOUNDS)


def _rsqrt16(x):
    i = lax.bitcast_convert_type(x, jnp.int32)
    i = jnp.int32(0x5F3759DF) - lax.shift_right_logical(i, 1)
    y = lax.bitcast_convert_type(i, jnp.float32)
    for _ in range(3):
        y = y * (1.5 - 0.5 * x * y * y)
    return y


@functools.partial(
    pl.kernel,
    out_type=jax.ShapeDtypeStruct((NPAD4,), jnp.float32),
    mesh=_mesh,
    compiler_params=_CP,
    scratch_types=[
        pltpu.VMEM((EN,), jnp.int32),
        pltpu.VMEM((EN,), jnp.float32),
        pltpu.VMEM((640,), jnp.float32),
        pltpu.VMEM_SHARED((NPAD4,), jnp.float32),
    ],
)
def _deg_dis_kernel(col_hbm, w_hbm, zeros_hbm, dis_hbm, colv, wv, dvm, deg):
    c = lax.axis_index("c")
    s = lax.axis_index("s")

    @pl.when(c == 0)
    def _():
        pltpu.sync_copy(col_hbm.at[s], colv)
        pltpu.sync_copy(w_hbm.at[s], wv)
        pltpu.sync_copy(zeros_hbm.at[pl.ds(s * 640, 640)],
                        deg.at[pl.ds(s * 640, 640)])
        plsc.subcore_barrier()
        pltpu.sync_copy(wv, deg.at[colv], add=True)
        plsc.subcore_barrier()
        pltpu.sync_copy(deg.at[pl.ds(s * 640, 640)], dvm)

        def body(g, carry):
            d16 = dvm[pl.ds(g * 16, 16)]
            y = _rsqrt16(d16)
            dvm[pl.ds(g * 16, 16)] = jnp.where(d16 > 0.0, y, 0.0)
            return carry

        lax.fori_loop(0, 40, body, 0)
        pltpu.sync_copy(dvm, dis_hbm.at[pl.ds(s * 640, 640)])


@functools.partial(
    pl.kernel,
    out_type=jax.ShapeDtypeStruct((32, EB), jnp.float32),
    mesh=_mesh,
    compiler_params=_CP,
    scratch_types=[
        pltpu.VMEM((EB,), jnp.int32),
        pltpu.VMEM((EB,), jnp.int32),
        pltpu.VMEM((EB,), jnp.float32),
        pltpu.VMEM((NPAD4,), jnp.float32),
        pltpu.VMEM((EB,), jnp.float32),
    ],
)
def _norm_kernel(row_hbm, col_hbm, w_hbm, dis_hbm, norm_hbm,
                 rv, cv, wv, disv, nv):
    c = lax.axis_index("c")
    s = lax.axis_index("s")
    wid = s * NCORES + c
    pltpu.sync_copy(row_hbm.at[wid], rv)
    pltpu.sync_copy(col_hbm.at[wid], cv)
    pltpu.sync_copy(w_hbm.at[wid], wv)
    pltpu.sync_copy(dis_hbm, disv)

    def body(g, carry):
        off = g * 16
        r16 = rv[pl.ds(off, 16)]
        c16 = cv[pl.ds(off, 16)]
        w16 = wv[pl.ds(off, 16)]
        dr = plsc.load_gather(disv, [r16])
        dc = plsc.load_gather(disv, [c16])
        nv[pl.ds(off, 16)] = dr * w16 * dc
        return carry

    lax.fori_loop(0, GB, body, 0)
    pltpu.sync_copy(nv, norm_hbm.at[wid])


@functools.partial(
    pl.kernel,
    out_type=(
        jax.ShapeDtypeStruct((NR, 32, SLOTR * 16), jnp.int32),
        jax.ShapeDtypeStruct((NR, 32, SLOTR * 16), jnp.int32),
        jax.ShapeDtypeStruct((NR, 32, SLOTR * 16), jnp.float32),
        jax.ShapeDtypeStruct((NR, 32, 16), jnp.int32),
    ),
    mesh=_mesh,
    compiler_params=_CP,
    scratch_types=[
        pltpu.VMEM((EB,), jnp.int32),
        pltpu.VMEM((EB,), jnp.int32),
        pltpu.VMEM((EB,), jnp.float32),
        pltpu.VMEM((SLOTR * 16,), jnp.int32),
        pltpu.VMEM((SLOTR * 16,), jnp.int32),
        pltpu.VMEM((SLOTR * 16,), jnp.float32),
        pltpu.VMEM((16,), jnp.int32),
    ],
)
def _compact_kernel(row_hbm, col_hbm, norm_hbm,
                    lrow_hbm, lcol_hbm, lnorm_hbm, lcnt_hbm,
                    cr, cc, cn, obr, obc, obn, cntv):
    c = lax.axis_index("c")
    s = lax.axis_index("s")
    iota = lax.iota(jnp.int32, 16)
    zi = jnp.zeros((16,), jnp.int32)
    zf = jnp.zeros((16,), jnp.float32)

    for p in range(2):
        rg = (s * NCORES + c) * 2 + p
        lo = rg * RR
        hi = lo + RR

        def slot(ch, carry0):
            pltpu.sync_copy(row_hbm.at[ch], cr)
            pltpu.sync_copy(col_hbm.at[ch], cc)
            pltpu.sync_copy(norm_hbm.at[ch], cn)

            def body(g, fill):
                off = g * 16
                r16 = cr[pl.ds(off, 16)]
                c16 = cc[pl.ds(off, 16)]
                n16 = cn[pl.ds(off, 16)]
                m = (c16 >= lo) & (c16 < hi)
                incl = plsc.cumsum(jnp.where(m, 1, 0))
                pos = fill + incl - 1
                plsc.store_scatter(obr, [pos], r16, mask=m)
                plsc.store_scatter(obc, [pos], c16 - lo, mask=m)
                plsc.store_scatter(obn, [pos], n16, mask=m)
                return fill + plsc.all_reduce_population_count(m)

            fill = lax.fori_loop(0, GB, body, zi)
            pad = jnp.bitwise_and(16 - jnp.bitwise_and(fill, 15), 15)
            m = iota < pad
            pos = fill + plsc.cumsum(jnp.where(m, 1, 0)) - 1
            plsc.store_scatter(obr, [pos], zi, mask=m)
            plsc.store_scatter(obc, [pos], zi, mask=m)
            plsc.store_scatter(obn, [pos], zf, mask=m)
            fillp = fill + plsc.all_reduce_population_count(m)
            pltpu.sync_copy(obr, lrow_hbm.at[rg, ch])
            pltpu.sync_copy(obc, lcol_hbm.at[rg, ch])
            pltpu.sync_copy(obn, lnorm_hbm.at[rg, ch])
            cntv[...] = lax.shift_right_logical(fillp, 4)
            pltpu.sync_copy(cntv, lcnt_hbm.at[rg, ch])
            return carry0

        lax.fori_loop(0, 32, slot, 0)


def _narrow_hops(h0_hbm, inits, outs, row_hbm, col_hbm, norm_hbm,
                 hv, msg, ivh, rv, cv, nvv, acc, s, write_last_only):
    pltpu.sync_copy(h0_hbm, hv)

    for k in range(3):
        pltpu.sync_copy(inits[k].at[pl.ds(s * 2560, 2560)],
                        acc.at[pl.ds(s * 2560, 2560)])
        plsc.subcore_barrier()
        for h in range(2):
            pltpu.sync_copy(row_hbm.at[s, h], rv)
            pltpu.sync_copy(col_hbm.at[s, h], cv)
            pltpu.sync_copy(norm_hbm.at[s, h], nvv)

            def body(g, carry):
                off = g * 16
                r16 = rv[pl.ds(off, 16)]
                c16 = cv[pl.ds(off, 16)]
                n16 = nvv[pl.ds(off, 16)]
                r4 = r16 * 4
                c4 = c16 * 4
                e4 = (off + lax.iota(jnp.int32, 16)) * 4
                for f in range(4):
                    vals = plsc.load_gather(hv, [r4 + f]) * n16
                    plsc.store_scatter(msg, [e4 + f], vals)
                    plsc.store_scatter(ivh, [e4 + f], c4 + f)
                return carry

            lax.fori_loop(0, GCH, body, 0, unroll=2)
            pltpu.sync_copy(msg, acc.at[ivh], add=True)
        plsc.subcore_barrier()
        if k < 2:
            pltpu.sync_copy(acc, hv)
        if (not write_last_only) or k == 2:
            pltpu.sync_copy(acc.at[pl.ds(s * 2560, 2560)],
                            outs[k].at[pl.ds(s * 2560, 2560)])
        plsc.subcore_barrier()


_NARROW_SCRATCH = [
    pltpu.VMEM((NPAD4 * 4,), jnp.float32),
    pltpu.VMEM((MCH,), jnp.float32),
    pltpu.VMEM((MCH,), jnp.int32),
    pltpu.VMEM((ECH,), jnp.int32),
    pltpu.VMEM((ECH,), jnp.int32),
    pltpu.VMEM((ECH,), jnp.float32),
    pltpu.VMEM_SHARED((NPAD4 * 4,), jnp.float32),
]


@functools.partial(
    pl.kernel,
    out_type=tuple(jax.ShapeDtypeStruct((NPAD4 * 4,), jnp.float32)
                   for _ in range(3)),
    mesh=_mesh,
    compiler_params=_CP,
    scratch_types=_NARROW_SCRATCH,
)
def _layer1_prop_kernel(x_hbm, zeros_hbm, row_hbm, col_hbm, norm_hbm,
                        p1, p2, p3, hv, msg, ivh, rv, cv, nvv, acc):
    c = lax.axis_index("c")
    s = lax.axis_index("s")

    @pl.when(c == 0)
    def _():
        _narrow_hops(x_hbm, [zeros_hbm] * 3, [p1, p2, p3],
                     row_hbm, col_hbm, norm_hbm,
                     hv, msg, ivh, rv, cv, nvv, acc, s, False)


@functools.partial(
    pl.kernel,
    out_type=jax.ShapeDtypeStruct((NPAD4 * 4,), jnp.float32),
    mesh=_mesh,
    compiler_params=_CP,
    scratch_types=_NARROW_SCRATCH,
)
def _layer3_prop_kernel(t0, t1, t2, t3, row_hbm, col_hbm, norm_hbm,
                        out, hv, msg, ivh, rv, cv, nvv, acc):
    c = lax.axis_index("c")
    s = lax.axis_index("s")

    @pl.when(c == 0)
    def _():
        _narrow_hops(t3, [t2, t1, t0], [out, out, out],
                     row_hbm, col_hbm, norm_hbm,
                     hv, msg, ivh, rv, cv, nvv, acc, s, True)


@functools.partial(
    pl.kernel,
    out_type=jax.ShapeDtypeStruct((NR * RR * H,), jnp.float32),
    mesh=_mesh,
    compiler_params=_CP,
    scratch_types=[
        pltpu.VMEM((1024,), jnp.int32),
        pltpu.VMEM((1024,), jnp.int32),
        pltpu.VMEM((1024,), jnp.float32),
        pltpu.VMEM((16, H), jnp.float32),
        pltpu.VMEM((16, H), jnp.float32),
        pltpu.VMEM((16,), jnp.int32),
        pltpu.VMEM((ACC1,), jnp.float32),
        pltpu.SemaphoreType.DMA,
        pltpu.SemaphoreType.DMA,
    ],
)
def _wide_prop_kernel(h_hbm, lrow_hbm, lcol_hbm, lnorm_hbm, lcnt_hbm,
                      zeros_hbm, out_hbm,
                      cbr, cbc, cbn, rows0, rows1, cntv, acc1, sem0, sem1):
    c = lax.axis_index("c")
    s = lax.axis_index("s")
    iota = lax.iota(jnp.int32, 16)

    def start(gv, buf, sem_):
        pltpu.async_copy(h_hbm.at[cbr.at[pl.ds(gv * 16, 16)]], buf, sem_)

    def waitbuf(buf, sem_):
        pltpu.make_async_copy(h_hbm.at[pl.ds(0, 16)], buf, sem_).wait()

    def do_group(gv, buf):
        off = gv * 16
        cl16 = cbc[pl.ds(off, 16)]
        n16 = cbn[pl.ds(off, 16)]

        @plsc.parallel_loop(0, 16, unroll=4)
        def _edge(e):
            nb = _blane(n16, e)
            bidx = _blane(cl16, e) * H + iota
            for j in range(H // 16):
                plsc.addupdate_scatter(
                    acc1, [bidx + j * 16], buf[e, pl.ds(j * 16, 16)] * nb)

    def prange(p, carryp):
        rg = (s * NCORES + c) * 2 + p
        pltpu.sync_copy(zeros_hbm, acc1)

        def slot(ch, carry0):
            pltpu.sync_copy(lcnt_hbm.at[rg, ch], cntv)
            trips = jnp.max(cntv[...])
            nco = lax.shift_right_logical(trips + 63, 6)

            def chunk(co, carry):
                base = co * 64
                pltpu.sync_copy(lrow_hbm.at[rg, ch, pl.ds(base * 16, 1024)], cbr)
                pltpu.sync_copy(lcol_hbm.at[rg, ch, pl.ds(base * 16, 1024)], cbc)
                pltpu.sync_copy(lnorm_hbm.at[rg, ch, pl.ds(base * 16, 1024)], cbn)
                nin = jnp.minimum(64, trips - base)

                @pl.when(nin > 0)
                def _():
                    start(0, rows0, sem0)

                @pl.when(nin > 1)
                def _():
                    start(1, rows1, sem1)

                def body2(i, carry2):
                    g0 = 2 * i
                    g1 = g0 + 1
                    waitbuf(rows0, sem0)
                    do_group(g0, rows0)

                    @pl.when(g0 + 2 < nin)
                    def _():
                        start(g0 + 2, rows0, sem0)

                    @pl.when(g1 < nin)
                    def _():
                        waitbuf(rows1, sem1)
                        do_group(g1, rows1)

                        @pl.when(g1 + 2 < nin)
                        def __():
                            start(g1 + 2, rows1, sem1)

                    return carry2

                lax.fori_loop(0, lax.shift_right_logical(nin + 1, 1), body2, 0)
                return carry

            lax.fori_loop(0, nco, chunk, 0)
            return carry0

        lax.fori_loop(0, 32, slot, 0)
        pltpu.sync_copy(acc1, out_hbm.at[pl.ds(rg * ACC1, ACC1)])
        return carryp

    lax.fori_loop(0, 2, prange, 0)


_PREC = lax.Precision.HIGHEST


def _elu(x):
    return jnp.where(x > 0.0, x, jnp.exp(jnp.minimum(x, 0.0)) - 1.0)


def _tc_lin_stats(xcat, w, b):
    n, fin = xcat.shape

    def body(x_ref, w_ref, b_ref, out_ref, st_ref):
        i = pl.program_id(0)
        h = jnp.dot(x_ref[...], w_ref[...],
                    preferred_element_type=jnp.float32, precision=_PREC)
        h = _elu(h + b_ref[...])
        out_ref[...] = h

        @pl.when(i == 0)
        def _():
            st_ref[...] = jnp.zeros_like(st_ref)

        st_ref[0:1, :] += jnp.sum(h, axis=0, keepdims=True)
        st_ref[1:2, :] += jnp.sum(h * h, axis=0, keepdims=True)

    return pl.pallas_call(
        body,
        grid=(10,),
        in_specs=[
            pl.BlockSpec((n // 10, fin), lambda i: (i, 0)),
            pl.BlockSpec((fin, H), lambda i: (0, 0)),
            pl.BlockSpec((H,), lambda i: (0,)),
        ],
        out_specs=[
            pl.BlockSpec((n // 10, H), lambda i: (i, 0)),
            pl.BlockSpec((8, H), lambda i: (0, 0)),
        ],
        out_shape=[
            jax.ShapeDtypeStruct((n, H), jnp.float32),
            jax.ShapeDtypeStruct((8, H), jnp.float32),
        ],
    )(xcat, w, b)


def _tc_lin4_stats(h1, z1, z2, z3, w2, b):
    n = h1.shape[0]

    def body(a_ref, b_ref_, c_ref, d_ref, w_ref, bias_ref, out_ref, st_ref):
        i = pl.program_id(0)
        h = jnp.dot(a_ref[...], w_ref[0],
                    preferred_element_type=jnp.float32, precision=_PREC)
        h += jnp.dot(b_ref_[...], w_ref[1],
                     preferred_element_type=jnp.float32, precision=_PREC)
        h += jnp.dot(c_ref[...], w_ref[2],
                     preferred_element_type=jnp.float32, precision=_PREC)
        h += jnp.dot(d_ref[...], w_ref[3],
                     preferred_element_type=jnp.float32, precision=_PREC)
        h = _elu(h + bias_ref[...])
        out_ref[...] = h

        @pl.when(i == 0)
        def _():
            st_ref[...] = jnp.zeros_like(st_ref)

        st_ref[0:1, :] += jnp.sum(h, axis=0, keepdims=True)
        st_ref[1:2, :] += jnp.sum(h * h, axis=0, keepdims=True)

    blk = pl.BlockSpec((n // 10, H), lambda i: (i, 0))
    return pl.pallas_call(
        body,
        grid=(10,),
        in_specs=[blk, blk, blk, blk,
                  pl.BlockSpec((4, H, H), lambda i: (0, 0, 0)),
                  pl.BlockSpec((H,), lambda i: (0,))],
        out_specs=[blk, pl.BlockSpec((8, H), lambda i: (0, 0))],
        out_shape=[
            jax.ShapeDtypeStruct((n, H), jnp.float32),
            jax.ShapeDtypeStruct((8, H), jnp.float32),
        ],
    )(h1, z1, z2, z3, w2, b)


def _tc_graph_norm(hpre, stats, w, b, ms):
    n = hpre.shape[0]

    def body(h_ref, st_ref, w_ref, b_ref, ms_ref, out_ref):
        m = st_ref[0:1, :] * (1.0 / n)
        s2 = st_ref[1:2, :] * (1.0 / n)
        msv = ms_ref[...]
        var = s2 - 2.0 * msv * m * m + msv * msv * m * m
        inv = lax.rsqrt(var + 1e-5)
        out_ref[...] = w_ref[...] * (h_ref[...] - m * msv) * inv + b_ref[...]

    return pl.pallas_call(
        body,
        grid=(10,),
        in_specs=[
            pl.BlockSpec((n // 10, H), lambda i: (i, 0)),
            pl.BlockSpec((8, H), lambda i: (0, 0)),
            pl.BlockSpec((H,), lambda i: (0,)),
            pl.BlockSpec((H,), lambda i: (0,)),
            pl.BlockSpec((H,), lambda i: (0,)),
        ],
        out_specs=pl.BlockSpec((n // 10, H), lambda i: (i, 0)),
        out_shape=jax.ShapeDtypeStruct((n, H), jnp.float32),
    )(hpre, stats, w, b, ms)


def _tc_final_lin(h2, w3r, b3r):
    n = h2.shape[0]

    def body(h_ref, w_ref, b_ref, out_ref):
        out_ref[...] = jnp.dot(
            h_ref[...], w_ref[...],
            preferred_element_type=jnp.float32, precision=_PREC) + b_ref[...]

    return pl.pallas_call(
        body,
        grid=(10,),
        in_specs=[
            pl.BlockSpec((n // 10, H), lambda i: (i, 0)),
            pl.BlockSpec((H, 16), lambda i: (0, 0)),
            pl.BlockSpec((16,), lambda i: (0,)),
        ],
        out_specs=pl.BlockSpec((n // 10, 16), lambda i: (i, 0)),
        out_shape=jax.ShapeDtypeStruct((n, 16), jnp.float32),
    )(h2, w3r, b3r)


def kernel(x, edge_index, weight, W1, b1, W2, b2, W3, b3,
           gn1_w, gn1_b, gn1_ms, gn2_w, gn2_b, gn2_ms):
    i32 = jnp.int32
    f32 = jnp.float32
    pad_e = EPAD - E
    rowp = jnp.concatenate([edge_index[0], jnp.zeros((pad_e,), i32)])
    colp = jnp.concatenate([edge_index[1], jnp.zeros((pad_e,), i32)])
    wp = jnp.concatenate([weight, jnp.zeros((pad_e,), f32)])

    zeros1 = jnp.zeros((NPAD4,), f32)
    zeros4 = jnp.zeros((NPAD4 * 4,), f32)
    zerosw = jnp.zeros((ACC1,), f32)

    dis = _deg_dis_kernel(colp.reshape(NT, EN), wp.reshape(NT, EN), zeros1)
    norm2 = _norm_kernel(rowp.reshape(32, EB), colp.reshape(32, EB),
                         wp.reshape(32, EB), dis)

    lrow, lcol, lnorm, lcnt = _compact_kernel(
        rowp.reshape(32, EB), colp.reshape(32, EB), norm2)

    rowN = rowp.reshape(NT, 2, ECH)
    colN = colp.reshape(NT, 2, ECH)
    normN = norm2.reshape(NT, 2, ECH)

    xflat = jnp.pad(x, ((0, NPAD4 - N), (0, 0))).reshape(-1)
    p1, p2, p3 = _layer1_prop_kernel(xflat, zeros4, rowN, colN, normN)

    xcat = jnp.concatenate(
        [x, p1.reshape(NPAD4, 4)[:N], p2.reshape(NPAD4, 4)[:N],
         p3.reshape(NPAD4, 4)[:N]], axis=1)
    h1pre, st1 = _tc_lin_stats(xcat, W1.reshape(4 * F_IN, H), b1)
    h1 = _tc_graph_norm(h1pre, st1, gn1_w, gn1_b, gn1_ms)

    h1p = jnp.pad(h1, ((0, NPADW - N), (0, 0)))
    z1r = _wide_prop_kernel(h1p, lrow, lcol, lnorm, lcnt,
                            zerosw).reshape(NPADW, H)
    z2r = _wide_prop_kernel(z1r, lrow, lcol, lnorm, lcnt,
                            zerosw).reshape(NPADW, H)
    z3r = _wide_prop_kernel(z2r, lrow, lcol, lnorm, lcnt,
                            zerosw).reshape(NPADW, H)

    h2pre, st2 = _tc_lin4_stats(h1, z1r[:N], z2r[:N], z3r[:N], W2, b2)
    h2 = _tc_graph_norm(h2pre, st2, gn2_w, gn2_b, gn2_ms)

    w3r = jnp.transpose(W3, (1, 0, 2)).reshape(H, 16)
    b3r = jnp.concatenate([b3, jnp.zeros((12,), f32)])
    t = _tc_final_lin(h2, w3r, b3r)
    tp = jnp.pad(t, ((0, NPAD4 - N), (0, 0)))
    t0 = tp[:, 0:4].reshape(-1)
    t1 = tp[:, 4:8].reshape(-1)
    t2 = tp[:, 8:12].reshape(-1)
    t3 = tp[:, 12:16].reshape(-1)

    out = _layer3_prop_kernel(t0, t1, t2, t3, rowN, colN, normN)
    return out.reshape(NPAD4, 4)[:N]
